# Initial kernel scaffold; baseline (speedup 1.0000x reference)
import jax
import jax.numpy as jnp
from jax.experimental import pallas as pl


def kernel(x, edge_index, W1_rel, b1, W1_root, W2_rel, b2, W2_root):
    raise NotImplementedError("write your pallas kernel here")


# speedup vs baseline: 11.5272x; 11.5272x over previous
import functools

import jax
import jax.numpy as jnp
from jax import lax
from jax.experimental import pallas as pl
from jax.experimental.pallas import tpu as pltpu
from jax.experimental.pallas import tpu_sc as plsc

N = 10000
E = 320000
D = 128
H = 16
C = 10

_NW = 32
_EPW = E // _NW
_B = 128
_K = (_EPW + _B - 1) // _B
_K = -(-_EPW // _B)
_EPW_PAD = _K * _B
_ACC_ROWS = 10112
_ZROWS = _ACC_ROWS // 16
_OROWS = _ACC_ROWS // 16


def _seg_body(table_hbm, src_hbm, dst_hbm, zeros_hbm, out_hbm,
              src_v, dst_v, rows_v, acc_sh, sem):
    c = lax.axis_index("c")
    s = lax.axis_index("s")
    wid = c * 16 + s
    pltpu.sync_copy(src_hbm.at[wid], src_v)
    pltpu.sync_copy(dst_hbm.at[wid], dst_v)
    pltpu.sync_copy(zeros_hbm, acc_sh.at[pl.ds(s * _ZROWS, _ZROWS)])
    plsc.subcore_barrier()

    def chunk(j, carry):
        pltpu.async_copy(table_hbm.at[src_v.at[j]], rows_v, sem).wait()
        pltpu.sync_copy(rows_v, acc_sh.at[dst_v.at[j]], add=True)
        return carry

    lax.fori_loop(0, _K, chunk, 0)
    plsc.subcore_barrier()
    pltpu.sync_copy(acc_sh.at[pl.ds(s * _OROWS, _OROWS)],
                    out_hbm.at[c, pl.ds(s * _OROWS, _OROWS)])


_seg_partial = functools.partial(
    pl.kernel,
    out_type=jax.ShapeDtypeStruct((2, _ACC_ROWS, H), jnp.float32),
    mesh=plsc.VectorSubcoreMesh(core_axis_name="c", subcore_axis_name="s"),
    compiler_params=pltpu.CompilerParams(use_tc_tiling_on_sc=False),
    scratch_types=[
        pltpu.VMEM((_K, _B), jnp.int32),
        pltpu.VMEM((_K, _B), jnp.int32),
        pltpu.VMEM((_B, H), jnp.float32),
        pltpu.VMEM_SHARED((_ACC_ROWS, H), jnp.float32),
        pltpu.SemaphoreType.DMA,
    ],
)(_seg_body)


_BLK = 2000


def _proj_body(x_ref, w_ref, b_ref, o_ref):
    o_ref[...] = lax.dot_general(
        x_ref[...], w_ref[...], (((1,), (1,)), ((), ())),
        preferred_element_type=jnp.float32) + b_ref[...]


def _proj(xx, wcat, bcat):
    n, d = xx.shape
    m = wcat.shape[0]
    return pl.pallas_call(
        _proj_body,
        grid=(n // _BLK,),
        in_specs=[
            pl.BlockSpec((_BLK, d), lambda i: (i, 0)),
            pl.BlockSpec((m, d), lambda i: (0, 0)),
            pl.BlockSpec((1, m), lambda i: (0, 0)),
        ],
        out_specs=pl.BlockSpec((_BLK, m), lambda i: (i, 0)),
        out_shape=jax.ShapeDtypeStruct((n, m), jnp.float32),
    )(xx, wcat, bcat)


def _mid_body(pa_ref, r_ref, w_ref, b_ref, o_ref):
    agg = pa_ref[0] + pa_ref[1]
    h = jnp.maximum(agg + r_ref[...], 0.0)
    o_ref[...] = lax.dot_general(
        h, w_ref[...], (((1,), (1,)), ((), ())),
        preferred_element_type=jnp.float32) + b_ref[...]


def _mid(partial1, r1, wcat, bcat):
    m = wcat.shape[0]
    return pl.pallas_call(
        _mid_body,
        grid=(N // _BLK,),
        in_specs=[
            pl.BlockSpec((2, _BLK, H), lambda i: (0, i, 0)),
            pl.BlockSpec((_BLK, H), lambda i: (i, 0)),
            pl.BlockSpec((m, H), lambda i: (0, 0)),
            pl.BlockSpec((1, m), lambda i: (0, 0)),
        ],
        out_specs=pl.BlockSpec((_BLK, m), lambda i: (i, 0)),
        out_shape=jax.ShapeDtypeStruct((N, m), jnp.float32),
    )(partial1, r1, wcat, bcat)


def _final_body(pa_ref, r_ref, o_ref):
    o = pa_ref[0] + pa_ref[1] + r_ref[...]
    mask = lax.broadcasted_iota(jnp.int32, o.shape, 1) < C
    neg = jnp.where(mask, o, -jnp.inf)
    m = jnp.max(neg, axis=1, keepdims=True)
    e = jnp.where(mask, jnp.exp(o - m), 0.0)
    ssum = jnp.sum(e, axis=1, keepdims=True)
    o_ref[...] = o - m - jnp.log(ssum)


def _final(partial2, r2):
    return pl.pallas_call(
        _final_body,
        grid=(N // _BLK,),
        in_specs=[
            pl.BlockSpec((2, _BLK, H), lambda i: (0, i, 0)),
            pl.BlockSpec((_BLK, H), lambda i: (i, 0)),
        ],
        out_specs=pl.BlockSpec((_BLK, H), lambda i: (i, 0)),
        out_shape=jax.ShapeDtypeStruct((N, H), jnp.float32),
    )(partial2, r2)


def kernel(x, edge_index, W1_rel, b1, W1_root, W2_rel, b2, W2_root):
    src = edge_index[0]
    dst = edge_index[1]
    pad = _EPW_PAD - _EPW
    srcw = jnp.pad(src.reshape(_NW, _EPW), ((0, 0), (0, pad)))
    dstw = jnp.pad(dst.reshape(_NW, _EPW), ((0, 0), (0, pad)),
                   constant_values=N)
    srcw = srcw.reshape(_NW, _K, _B)
    dstw = dstw.reshape(_NW, _K, _B)
    zrows = jnp.zeros((_ZROWS, H), jnp.float32)

    wcat1 = jnp.concatenate([W1_rel, W1_root], axis=0)
    bcat1 = jnp.concatenate([jnp.zeros_like(b1), b1]).reshape(1, 2 * H)
    out1 = _proj(x, wcat1, bcat1)
    p1 = out1[:, :H]
    r1 = out1[:, H:]
    partial1 = _seg_partial(p1, srcw, dstw, zrows)[:, :N]

    w2rel = jnp.pad(W2_rel, ((0, H - C), (0, 0)))
    w2root = jnp.pad(W2_root, ((0, H - C), (0, 0)))
    wcat2 = jnp.concatenate([w2rel, w2root], axis=0)
    b2p = jnp.pad(b2, (0, H - C))
    bcat2 = jnp.concatenate([jnp.zeros_like(b2p), b2p]).reshape(1, 2 * H)
    out2 = _mid(partial1, r1, wcat2, bcat2)
    p2 = out2[:, :H]
    r2 = out2[:, H:]
    partial2 = _seg_partial(p2, srcw, dstw, zrows)[:, :N]

    out16 = _final(partial2, r2)
    return out16[:, :C]
